# baseline (speedup 1.0000x reference)
import functools

import jax
import jax.numpy as jnp
from jax import lax
from jax.experimental import pallas as pl
from jax.experimental.pallas import tpu as pltpu
from jax.experimental.pallas import tpu_sc as plsc

N = 10000
E = 320000
D = 128

NC = 2
NS = 16
NW = NC * NS
CHUNK = 128
NCH = 80
EDGES_PER_TILE = CHUNK * NCH
E_PAD = EDGES_PER_TILE * NW
N_PAD = 10240
ROWS_PER_TILE = N_PAD // NS
DUMMY = N
NV = EDGES_PER_TILE // 16
RB = 1024
GRID = N_PAD // RB

def _sc_degree_body(dst_hbm, out_hbm, dst_v, hist, sem):
    c = lax.axis_index("c")
    s = lax.axis_index("s")
    wid = s * NC + c
    pltpu.sync_copy(dst_hbm.at[wid], dst_v)

    def zero(i, carry):
        hist[pl.ds(i * 16, 16)] = jnp.zeros((16,), jnp.float32)
        return carry

    lax.fori_loop(0, N_PAD // 16, zero, 0)
    ones = jnp.ones((16,), jnp.float32)

    def body(i, carry):
        plsc.addupdate_scatter(hist, [dst_v[i]], ones)
        return carry

    lax.fori_loop(0, NV, body, 0)
    pltpu.sync_copy(hist, out_hbm.at[wid])


def _sc_scatter_rows_body(g_hbm, packed_hbm, zeros_hbm, out_hbm,
                          packed_v, sidx0, sidx1, didx0, didx1,
                          rows0, rows1, acc, sem0, sem1):
    c = lax.axis_index("c")
    s = lax.axis_index("s")
    wid = s * NC + c
    pltpu.sync_copy(zeros_hbm.at[pl.ds(s * ROWS_PER_TILE, ROWS_PER_TILE)],
                    acc.at[pl.ds(s * ROWS_PER_TILE, ROWS_PER_TILE)])
    pltpu.sync_copy(packed_hbm.at[wid], packed_v)
    plsc.subcore_barrier()

    def unpack(i, sidx, didx):
        def u(v, carry):
            pk = packed_v[i, pl.ds(v * 16, 16)]
            sidx[pl.ds(v * 16, 16)] = jnp.bitwise_and(pk, 65535)
            didx[pl.ds(v * 16, 16)] = lax.shift_right_logical(pk, 16)
            return carry
        lax.fori_loop(0, CHUNK // 16, u, 0)

    unpack(0, sidx0, didx0)
    unpack(1, sidx1, didx1)
    pltpu.async_copy(g_hbm.at[sidx0], rows0, sem0)
    pltpu.async_copy(g_hbm.at[sidx1], rows1, sem1)

    def body(j, carry):
        i0 = 2 * j
        i1 = i0 + 1
        pltpu.make_async_copy(g_hbm.at[sidx0], rows0, sem0).wait()
        pltpu.sync_copy(rows0, acc.at[didx0], add=True)
        unpack(i0 + 2, sidx0, didx0)
        pltpu.async_copy(g_hbm.at[sidx0], rows0, sem0)
        pltpu.make_async_copy(g_hbm.at[sidx1], rows1, sem1).wait()
        pltpu.sync_copy(rows1, acc.at[didx1], add=True)
        unpack(i1 + 2, sidx1, didx1)
        pltpu.async_copy(g_hbm.at[sidx1], rows1, sem1)
        return carry

    lax.fori_loop(0, NCH // 2, body, 0)
    pltpu.make_async_copy(g_hbm.at[sidx0], rows0, sem0).wait()
    pltpu.make_async_copy(g_hbm.at[sidx1], rows1, sem1).wait()
    plsc.subcore_barrier()
    pltpu.sync_copy(acc.at[pl.ds(s * ROWS_PER_TILE, ROWS_PER_TILE)],
                    out_hbm.at[c, pl.ds(s * ROWS_PER_TILE, ROWS_PER_TILE)])


@functools.cache
def _sc_kernels():
    mesh = plsc.VectorSubcoreMesh(
        core_axis_name="c", subcore_axis_name="s",
        num_cores=NC, num_subcores=NS)
    sc_degree = pl.kernel(
        _sc_degree_body,
        out_type=jax.ShapeDtypeStruct((NW, N_PAD), jnp.float32),
        mesh=mesh,
        compiler_params=pltpu.CompilerParams(needs_layout_passes=False),
        scratch_types=[
            pltpu.VMEM((NV, 16), jnp.int32),
            pltpu.VMEM((N_PAD,), jnp.float32),
            pltpu.SemaphoreType.DMA,
        ],
    )
    sc_scatter_rows = pl.kernel(
        _sc_scatter_rows_body,
        out_type=jax.ShapeDtypeStruct((NC, N_PAD, D), jnp.float32),
        mesh=mesh,
        scratch_types=[
            pltpu.VMEM((NCH + 2, CHUNK), jnp.int32),
            pltpu.VMEM((CHUNK,), jnp.int32),
            pltpu.VMEM((CHUNK,), jnp.int32),
            pltpu.VMEM((CHUNK,), jnp.int32),
            pltpu.VMEM((CHUNK,), jnp.int32),
            pltpu.VMEM((CHUNK, D), jnp.float32),
            pltpu.VMEM((CHUNK, D), jnp.float32),
            pltpu.VMEM_SHARED((N_PAD, D), jnp.float32),
            pltpu.SemaphoreType.DMA,
            pltpu.SemaphoreType.DMA,
        ],
    )
    return sc_degree, sc_scatter_rows


def _deg_scale(ht):
    return lax.rsqrt(jnp.sum(ht, axis=1, keepdims=True) + 1.0)


def _t1_body(x_ref, w_ref, ht_ref, g_ref):
    d = _deg_scale(ht_ref[...])
    h = jnp.dot(x_ref[...], w_ref[...], preferred_element_type=jnp.float32)
    g_ref[...] = h * d


def _t2_body(a0_ref, a1_ref, g_ref, ht_ref, b_ref, w_ref, out_ref):
    d = _deg_scale(ht_ref[...])
    z = d * (a0_ref[...] + a1_ref[...] + g_ref[...]) + b_ref[...]
    r = jnp.maximum(z, 0.0)
    out_ref[...] = jnp.dot(r, w_ref[...], preferred_element_type=jnp.float32) * d


def _t3_body(a0_ref, a1_ref, g_ref, ht_ref, b_ref,
             h_ref, logp_ref, pred_ref):
    d = _deg_scale(ht_ref[...])
    z = d * (a0_ref[...] + a1_ref[...] + g_ref[...]) + b_ref[...]
    h_ref[...] = z
    m = jnp.max(z, axis=1, keepdims=True)
    lse = m + jnp.log(jnp.sum(jnp.exp(z - m), axis=1, keepdims=True))
    logp_ref[...] = z - lse
    idx = lax.broadcasted_iota(jnp.int32, z.shape, 1)
    pred = jnp.min(jnp.where(z == m, idx, jnp.int32(2**30)), axis=1)
    pred_ref[...] = pred[:, None]


def _row_spec(width=D):
    return pl.BlockSpec((RB, width), lambda i: (i, 0))


def _full_spec(shape):
    return pl.BlockSpec(shape, lambda i: (0,) * len(shape))


def kernel(x, edge_index, W1, b1, W2, b2):
    src = edge_index[0]
    dst = edge_index[1]
    pad = E_PAD - E
    src_p = jnp.concatenate([src, jnp.zeros((pad,), jnp.int32)]).reshape(
        NW, NCH, CHUNK)
    dst_p = jnp.concatenate([dst, jnp.full((pad,), DUMMY, jnp.int32)]).reshape(
        NW, NCH, CHUNK)
    packed = src_p + dst_p * 65536
    packed_sc = jnp.concatenate(
        [packed, jnp.full((NW, 2, CHUNK), DUMMY * 65536, jnp.int32)],
        axis=1)
    x_pad = jnp.pad(x, ((0, N_PAD - N), (0, 0)))
    zerosD = jnp.zeros((N_PAD, D), jnp.float32)
    b1r = b1.reshape(1, D)
    b2r = b2.reshape(1, D)

    sc_degree, sc_scatter_rows = _sc_kernels()
    hist = sc_degree(dst_p.reshape(NW, NV, 16))
    ht = hist.T

    g1 = pl.pallas_call(
        _t1_body,
        grid=(GRID,),
        in_specs=[_row_spec(), _full_spec((D, D)), _row_spec(NW)],
        out_specs=_row_spec(),
        out_shape=jax.ShapeDtypeStruct((N_PAD, D), jnp.float32),
    )(x_pad, W1, ht)

    acc1 = sc_scatter_rows(g1, packed_sc, zerosD)

    g2 = pl.pallas_call(
        _t2_body,
        grid=(GRID,),
        in_specs=[_row_spec(), _row_spec(), _row_spec(), _row_spec(NW),
                  _full_spec((1, D)), _full_spec((D, D))],
        out_specs=_row_spec(),
        out_shape=jax.ShapeDtypeStruct((N_PAD, D), jnp.float32),
    )(acc1[0], acc1[1], g1, ht, b1r, W2)

    acc2 = sc_scatter_rows(g2, packed_sc, zerosD)

    h_out, logp, pred = pl.pallas_call(
        _t3_body,
        grid=(GRID,),
        in_specs=[_row_spec(), _row_spec(), _row_spec(), _row_spec(NW),
                  _full_spec((1, D))],
        out_specs=[_row_spec(), _row_spec(), _row_spec(1)],
        out_shape=[
            jax.ShapeDtypeStruct((N_PAD, D), jnp.float32),
            jax.ShapeDtypeStruct((N_PAD, D), jnp.float32),
            jax.ShapeDtypeStruct((N_PAD, 1), jnp.int32),
        ],
    )(acc2[0], acc2[1], g2, ht, b2r)

    return (h_out[:N], logp[:N], pred[:N, 0])


# speedup vs baseline: 1.7856x; 1.7856x over previous
import functools

import jax
import jax.numpy as jnp
from jax import lax
from jax.experimental import pallas as pl
from jax.experimental.pallas import tpu as pltpu
from jax.experimental.pallas import tpu_sc as plsc

N = 10000
E = 320000
D = 128

NC = 2
NS = 16
NW = NC * NS
CHUNK = 128
NCH = 80
EDGES_PER_TILE = CHUNK * NCH
E_PAD = EDGES_PER_TILE * NW
N_PAD = 10240
ROWS_PER_TILE = N_PAD // NS
DUMMY = N
NV = EDGES_PER_TILE // 16
KSUP = 2
RB = 1024
GRID = N_PAD // RB

def _sc_degree_body(dst_hbm, out_hbm, dst_v, hist, sem):
    c = lax.axis_index("c")
    s = lax.axis_index("s")
    wid = s * NC + c
    pltpu.sync_copy(dst_hbm.at[wid], dst_v)

    def zero(i, carry):
        hist[pl.ds(i * 16, 16)] = jnp.zeros((16,), jnp.float32)
        return carry

    lax.fori_loop(0, N_PAD // 16, zero, 0)
    ones = jnp.ones((16,), jnp.float32)

    def body(i, carry):
        plsc.addupdate_scatter(hist, [dst_v[i]], ones)
        return carry

    lax.fori_loop(0, NV, body, 0)
    pltpu.sync_copy(hist, out_hbm.at[wid])


def _sc_scatter_rows_body(g_hbm, packed_hbm, zeros_hbm, out_hbm,
                          packed_v, sidx, didx, rows, acc, sem0):
    c = lax.axis_index("c")
    s = lax.axis_index("s")
    wid = s * NC + c
    pltpu.sync_copy(zeros_hbm.at[pl.ds(s * ROWS_PER_TILE, ROWS_PER_TILE)],
                    acc.at[pl.ds(s * ROWS_PER_TILE, ROWS_PER_TILE)])
    pltpu.sync_copy(packed_hbm.at[wid], packed_v)
    plsc.subcore_barrier()

    def body(j, carry):
        def u(v, carry2):
            i = j * KSUP + v // (CHUNK // 16)
            col = (v % (CHUNK // 16)) * 16
            pk = packed_v[i, pl.ds(col, 16)]
            sidx[pl.ds(v * 16, 16)] = jnp.bitwise_and(pk, 65535)
            didx[pl.ds(v * 16, 16)] = lax.shift_right_logical(pk, 16)
            return carry2

        lax.fori_loop(0, KSUP * (CHUNK // 16), u, 0)
        pltpu.async_copy(g_hbm.at[sidx], rows, sem0).wait()
        pltpu.sync_copy(rows, acc.at[didx], add=True)
        return carry

    lax.fori_loop(0, NCH // KSUP, body, 0)
    plsc.subcore_barrier()
    pltpu.sync_copy(acc.at[pl.ds(s * ROWS_PER_TILE, ROWS_PER_TILE)],
                    out_hbm.at[c, pl.ds(s * ROWS_PER_TILE, ROWS_PER_TILE)])


@functools.cache
def _sc_kernels():
    mesh = plsc.VectorSubcoreMesh(
        core_axis_name="c", subcore_axis_name="s",
        num_cores=NC, num_subcores=NS)
    sc_degree = pl.kernel(
        _sc_degree_body,
        out_type=jax.ShapeDtypeStruct((NW, N_PAD), jnp.float32),
        mesh=mesh,
        compiler_params=pltpu.CompilerParams(needs_layout_passes=False),
        scratch_types=[
            pltpu.VMEM((NV, 16), jnp.int32),
            pltpu.VMEM((N_PAD,), jnp.float32),
            pltpu.SemaphoreType.DMA,
        ],
    )
    sc_scatter_rows = pl.kernel(
        _sc_scatter_rows_body,
        out_type=jax.ShapeDtypeStruct((NC, N_PAD, D), jnp.float32),
        mesh=mesh,
        scratch_types=[
            pltpu.VMEM((NCH, CHUNK), jnp.int32),
            pltpu.VMEM((KSUP * CHUNK,), jnp.int32),
            pltpu.VMEM((KSUP * CHUNK,), jnp.int32),
            pltpu.VMEM((KSUP * CHUNK, D), jnp.float32),
            pltpu.VMEM_SHARED((N_PAD, D), jnp.float32),
            pltpu.SemaphoreType.DMA,
        ],
    )
    return sc_degree, sc_scatter_rows


def _deg_scale(ht):
    return lax.rsqrt(jnp.sum(ht, axis=1, keepdims=True) + 1.0)


def _t1_body(x_ref, w_ref, ht_ref, g_ref):
    d = _deg_scale(ht_ref[...])
    h = jnp.dot(x_ref[...], w_ref[...], preferred_element_type=jnp.float32)
    g_ref[...] = h * d


def _t2_body(a0_ref, a1_ref, g_ref, ht_ref, b_ref, w_ref, out_ref):
    d = _deg_scale(ht_ref[...])
    z = d * (a0_ref[...] + a1_ref[...] + g_ref[...]) + b_ref[...]
    r = jnp.maximum(z, 0.0)
    out_ref[...] = jnp.dot(r, w_ref[...], preferred_element_type=jnp.float32) * d


def _t3_body(a0_ref, a1_ref, g_ref, ht_ref, b_ref,
             h_ref, logp_ref, pred_ref):
    d = _deg_scale(ht_ref[...])
    z = d * (a0_ref[...] + a1_ref[...] + g_ref[...]) + b_ref[...]
    h_ref[...] = z
    m = jnp.max(z, axis=1, keepdims=True)
    lse = m + jnp.log(jnp.sum(jnp.exp(z - m), axis=1, keepdims=True))
    logp_ref[...] = z - lse
    idx = lax.broadcasted_iota(jnp.int32, z.shape, 1)
    pred = jnp.min(jnp.where(z == m, idx, jnp.int32(2**30)), axis=1)
    pred_ref[...] = pred[:, None]


def _row_spec(width=D):
    return pl.BlockSpec((RB, width), lambda i: (i, 0))


def _full_spec(shape):
    return pl.BlockSpec(shape, lambda i: (0,) * len(shape))


def kernel(x, edge_index, W1, b1, W2, b2):
    src = edge_index[0]
    dst = edge_index[1]
    pad = E_PAD - E
    src_p = jnp.concatenate([src, jnp.zeros((pad,), jnp.int32)]).reshape(
        NW, NCH, CHUNK)
    dst_p = jnp.concatenate([dst, jnp.full((pad,), DUMMY, jnp.int32)]).reshape(
        NW, NCH, CHUNK)
    packed_sc = src_p + dst_p * 65536
    x_pad = jnp.pad(x, ((0, N_PAD - N), (0, 0)))
    zerosD = jnp.zeros((N_PAD, D), jnp.float32)
    b1r = b1.reshape(1, D)
    b2r = b2.reshape(1, D)

    sc_degree, sc_scatter_rows = _sc_kernels()
    hist = sc_degree(dst_p.reshape(NW, NV, 16))
    ht = hist.T

    g1 = pl.pallas_call(
        _t1_body,
        grid=(GRID,),
        in_specs=[_row_spec(), _full_spec((D, D)), _row_spec(NW)],
        out_specs=_row_spec(),
        out_shape=jax.ShapeDtypeStruct((N_PAD, D), jnp.float32),
    )(x_pad, W1, ht)

    acc1 = sc_scatter_rows(g1, packed_sc, zerosD)

    g2 = pl.pallas_call(
        _t2_body,
        grid=(GRID,),
        in_specs=[_row_spec(), _row_spec(), _row_spec(), _row_spec(NW),
                  _full_spec((1, D)), _full_spec((D, D))],
        out_specs=_row_spec(),
        out_shape=jax.ShapeDtypeStruct((N_PAD, D), jnp.float32),
    )(acc1[0], acc1[1], g1, ht, b1r, W2)

    acc2 = sc_scatter_rows(g2, packed_sc, zerosD)

    h_out, logp, pred = pl.pallas_call(
        _t3_body,
        grid=(GRID,),
        in_specs=[_row_spec(), _row_spec(), _row_spec(), _row_spec(NW),
                  _full_spec((1, D))],
        out_specs=[_row_spec(), _row_spec(), _row_spec(1)],
        out_shape=[
            jax.ShapeDtypeStruct((N_PAD, D), jnp.float32),
            jax.ShapeDtypeStruct((N_PAD, D), jnp.float32),
            jax.ShapeDtypeStruct((N_PAD, 1), jnp.int32),
        ],
    )(acc2[0], acc2[1], g2, ht, b2r)

    return (h_out[:N], logp[:N], pred[:N, 0])


# speedup vs baseline: 1.9039x; 1.0662x over previous
import functools

import jax
import jax.numpy as jnp
from jax import lax
from jax.experimental import pallas as pl
from jax.experimental.pallas import tpu as pltpu
from jax.experimental.pallas import tpu_sc as plsc

N = 10000
E = 320000
D = 128

NC = 2
NS = 16
NW = NC * NS
CHUNK = 128
NCH = 80
EDGES_PER_TILE = CHUNK * NCH
E_PAD = EDGES_PER_TILE * NW
N_PAD = 10240
ROWS_PER_TILE = N_PAD // NS
DUMMY = N
NV = EDGES_PER_TILE // 16
KSUP = 2
SLOW_C = 1
NCH_S = 44
NCH_F = 2 * NCH - NCH_S
RB = 1024
GRID = N_PAD // RB

def _sc_degree_body(dst_hbm, out_hbm, dst_v, hist, sem):
    c = lax.axis_index("c")
    s = lax.axis_index("s")
    wid = s * NC + c
    pltpu.sync_copy(dst_hbm.at[wid], dst_v)

    def zero(i, carry):
        hist[pl.ds(i * 16, 16)] = jnp.zeros((16,), jnp.float32)
        return carry

    lax.fori_loop(0, N_PAD // 16, zero, 0)
    ones = jnp.ones((16,), jnp.float32)

    def body(i, carry):
        plsc.addupdate_scatter(hist, [dst_v[i]], ones)
        return carry

    lax.fori_loop(0, NV, body, 0)
    pltpu.sync_copy(hist, out_hbm.at[wid])


def _sc_scatter_rows_body(g_hbm, packed_hbm, zeros_hbm, out_hbm,
                          packed_v, sidx, didx, rows, acc, sem0):
    c = lax.axis_index("c")
    s = lax.axis_index("s")
    wid = s * NC + c
    pltpu.sync_copy(zeros_hbm.at[pl.ds(s * ROWS_PER_TILE, ROWS_PER_TILE)],
                    acc.at[pl.ds(s * ROWS_PER_TILE, ROWS_PER_TILE)])
    pltpu.sync_copy(packed_hbm.at[wid], packed_v)
    plsc.subcore_barrier()

    def body(j, carry):
        def u(v, carry2):
            i = j * KSUP + v // (CHUNK // 16)
            col = (v % (CHUNK // 16)) * 16
            pk = packed_v[i, pl.ds(col, 16)]
            sidx[pl.ds(v * 16, 16)] = jnp.bitwise_and(pk, 65535)
            didx[pl.ds(v * 16, 16)] = lax.shift_right_logical(pk, 16)
            return carry2

        lax.fori_loop(0, KSUP * (CHUNK // 16), u, 0)
        pltpu.async_copy(g_hbm.at[sidx], rows, sem0).wait()
        pltpu.sync_copy(rows, acc.at[didx], add=True)
        return carry

    n_mine = jnp.where(c == SLOW_C, NCH_S // KSUP, NCH_F // KSUP)
    lax.fori_loop(0, n_mine, body, 0)
    plsc.subcore_barrier()
    pltpu.sync_copy(acc.at[pl.ds(s * ROWS_PER_TILE, ROWS_PER_TILE)],
                    out_hbm.at[c, pl.ds(s * ROWS_PER_TILE, ROWS_PER_TILE)])


@functools.cache
def _sc_kernels():
    mesh = plsc.VectorSubcoreMesh(
        core_axis_name="c", subcore_axis_name="s",
        num_cores=NC, num_subcores=NS)
    sc_degree = pl.kernel(
        _sc_degree_body,
        out_type=jax.ShapeDtypeStruct((NW, N_PAD), jnp.float32),
        mesh=mesh,
        compiler_params=pltpu.CompilerParams(needs_layout_passes=False),
        scratch_types=[
            pltpu.VMEM((NV, 16), jnp.int32),
            pltpu.VMEM((N_PAD,), jnp.float32),
            pltpu.SemaphoreType.DMA,
        ],
    )
    sc_scatter_rows = pl.kernel(
        _sc_scatter_rows_body,
        out_type=jax.ShapeDtypeStruct((NC, N_PAD, D), jnp.float32),
        mesh=mesh,
        scratch_types=[
            pltpu.VMEM((NCH_F, CHUNK), jnp.int32),
            pltpu.VMEM((KSUP * CHUNK,), jnp.int32),
            pltpu.VMEM((KSUP * CHUNK,), jnp.int32),
            pltpu.VMEM((KSUP * CHUNK, D), jnp.float32),
            pltpu.VMEM_SHARED((N_PAD, D), jnp.float32),
            pltpu.SemaphoreType.DMA,
        ],
    )
    return sc_degree, sc_scatter_rows


def _deg_scale(ht):
    return lax.rsqrt(jnp.sum(ht, axis=1, keepdims=True) + 1.0)


def _t1_body(x_ref, w_ref, ht_ref, g_ref):
    d = _deg_scale(ht_ref[...])
    h = jnp.dot(x_ref[...], w_ref[...], preferred_element_type=jnp.float32)
    g_ref[...] = h * d


def _t2_body(a0_ref, a1_ref, g_ref, ht_ref, b_ref, w_ref, out_ref):
    d = _deg_scale(ht_ref[...])
    z = d * (a0_ref[...] + a1_ref[...] + g_ref[...]) + b_ref[...]
    r = jnp.maximum(z, 0.0)
    out_ref[...] = jnp.dot(r, w_ref[...], preferred_element_type=jnp.float32) * d


def _t3_body(a0_ref, a1_ref, g_ref, ht_ref, b_ref,
             h_ref, logp_ref, pred_ref):
    d = _deg_scale(ht_ref[...])
    z = d * (a0_ref[...] + a1_ref[...] + g_ref[...]) + b_ref[...]
    h_ref[...] = z
    m = jnp.max(z, axis=1, keepdims=True)
    lse = m + jnp.log(jnp.sum(jnp.exp(z - m), axis=1, keepdims=True))
    logp_ref[...] = z - lse
    idx = lax.broadcasted_iota(jnp.int32, z.shape, 1)
    pred = jnp.min(jnp.where(z == m, idx, jnp.int32(2**30)), axis=1)
    pred_ref[...] = pred[:, None]


def _row_spec(width=D):
    return pl.BlockSpec((RB, width), lambda i: (i, 0))


def _full_spec(shape):
    return pl.BlockSpec(shape, lambda i: (0,) * len(shape))


def kernel(x, edge_index, W1, b1, W2, b2):
    src = edge_index[0]
    dst = edge_index[1]
    pad = E_PAD - E
    src_p = jnp.concatenate([src, jnp.zeros((pad,), jnp.int32)]).reshape(
        NW, NCH, CHUNK)
    dst_p = jnp.concatenate([dst, jnp.full((pad,), DUMMY, jnp.int32)]).reshape(
        NW, NCH, CHUNK)
    packed_flat = (src_p + dst_p * 65536).reshape(NW * NCH, CHUNK)
    n_fast = NS * NCH_F
    a_fast = packed_flat[:n_fast].reshape(NS, NCH_F, CHUNK)
    a_slow = packed_flat[n_fast:].reshape(NS, NCH_S, CHUNK)
    a_slow = jnp.concatenate(
        [a_slow,
         jnp.full((NS, NCH_F - NCH_S, CHUNK), DUMMY * 65536, jnp.int32)],
        axis=1)
    pair = (a_slow, a_fast) if SLOW_C == 0 else (a_fast, a_slow)
    packed_sc = jnp.stack(pair, axis=1).reshape(NW, NCH_F, CHUNK)
    x_pad = jnp.pad(x, ((0, N_PAD - N), (0, 0)))
    zerosD = jnp.zeros((N_PAD, D), jnp.float32)
    b1r = b1.reshape(1, D)
    b2r = b2.reshape(1, D)

    sc_degree, sc_scatter_rows = _sc_kernels()
    hist = sc_degree(dst_p.reshape(NW, NV, 16))
    ht = hist.T

    g1 = pl.pallas_call(
        _t1_body,
        grid=(GRID,),
        in_specs=[_row_spec(), _full_spec((D, D)), _row_spec(NW)],
        out_specs=_row_spec(),
        out_shape=jax.ShapeDtypeStruct((N_PAD, D), jnp.float32),
    )(x_pad, W1, ht)

    acc1 = sc_scatter_rows(g1, packed_sc, zerosD)

    g2 = pl.pallas_call(
        _t2_body,
        grid=(GRID,),
        in_specs=[_row_spec(), _row_spec(), _row_spec(), _row_spec(NW),
                  _full_spec((1, D)), _full_spec((D, D))],
        out_specs=_row_spec(),
        out_shape=jax.ShapeDtypeStruct((N_PAD, D), jnp.float32),
    )(acc1[0], acc1[1], g1, ht, b1r, W2)

    acc2 = sc_scatter_rows(g2, packed_sc, zerosD)

    h_out, logp, pred = pl.pallas_call(
        _t3_body,
        grid=(GRID,),
        in_specs=[_row_spec(), _row_spec(), _row_spec(), _row_spec(NW),
                  _full_spec((1, D))],
        out_specs=[_row_spec(), _row_spec(), _row_spec(1)],
        out_shape=[
            jax.ShapeDtypeStruct((N_PAD, D), jnp.float32),
            jax.ShapeDtypeStruct((N_PAD, D), jnp.float32),
            jax.ShapeDtypeStruct((N_PAD, 1), jnp.int32),
        ],
    )(acc2[0], acc2[1], g2, ht, b2r)

    return (h_out[:N], logp[:N], pred[:N, 0])
